# baseline (speedup 1.0000x reference)
import jax
import jax.numpy as jnp
from jax.experimental import pallas as pl

NUM_PATCHES = 5
HIDDEN = 16
BATCH_BLOCK = 8


def _fill_body(o_ref):
    p = jax.lax.broadcasted_iota(jnp.int32, o_ref.shape, 1)
    o_ref[...] = jnp.maximum(p - (NUM_PATCHES - 1), 0).astype(jnp.float32)


def kernel(pixel_values, input_ids, labels):
    batch, seq_len = input_ids.shape
    total = seq_len + NUM_PATCHES
    return pl.pallas_call(
        _fill_body,
        grid=(batch // BATCH_BLOCK,),
        out_specs=pl.BlockSpec((BATCH_BLOCK, total, HIDDEN), lambda i: (i, 0, 0)),
        out_shape=jax.ShapeDtypeStruct((batch, total, HIDDEN), jnp.float32),
    )()


# speedup vs baseline: 1.4878x; 1.4878x over previous
import jax
import jax.numpy as jnp
from jax.experimental import pallas as pl
from jax.experimental.pallas import tpu as pltpu

NUM_PATCHES = 5
HIDDEN = 16
BLOCK_BATCH = 8
NUM_SEMS = 2


def _body(o_ref, block_v, *sems):
    w = jax.lax.broadcasted_iota(jnp.int32, block_v.shape, 1)
    p = jax.lax.shift_right_logical(w, HIDDEN.bit_length() - 1)
    block_v[...] = jnp.maximum(p - (NUM_PATCHES - 1), 0).astype(jnp.float32)
    batch = o_ref.shape[0]
    n = batch // BLOCK_BATCH
    copies = [
        pltpu.make_async_copy(
            block_v,
            o_ref.at[pl.ds(i * BLOCK_BATCH, BLOCK_BATCH)],
            sems[i % NUM_SEMS],
        )
        for i in range(n)
    ]
    for c in copies:
        c.start()
    for c in copies:
        c.wait()


def kernel(pixel_values, input_ids, labels):
    batch, seq_len = input_ids.shape
    total = seq_len + NUM_PATCHES
    flat = pl.pallas_call(
        _body,
        out_specs=pl.BlockSpec(memory_space=pl.ANY),
        out_shape=jax.ShapeDtypeStruct((batch, total * HIDDEN), jnp.float32),
        scratch_shapes=[pltpu.VMEM((BLOCK_BATCH, total * HIDDEN), jnp.float32)]
        + [pltpu.SemaphoreType.DMA] * NUM_SEMS,
    )()
    return jnp.reshape(flat, (batch, total, HIDDEN))
